# baseline (speedup 1.0000x reference)
import functools

import jax
import jax.numpy as jnp
from jax import lax
from jax.experimental import pallas as pl
from jax.experimental.pallas import tpu as pltpu
from jax.experimental.pallas import tpu_sc as plsc

F = 26
B = 16384
FD = 100000
NC, NS, L = 2, 16, 16
NW = NC * NS
BPW = B // NW
CHUNK = F * BPW
NJ = BPW // L


def kernel(x, W, bias):
    wf = W.reshape(-1)
    mesh = plsc.VectorSubcoreMesh(core_axis_name="c", subcore_axis_name="s")

    @functools.partial(
        pl.kernel,
        mesh=mesh,
        out_type=jax.ShapeDtypeStruct((B,), jnp.float32),
        compiler_params=pltpu.CompilerParams(needs_layout_passes=False),
        scratch_types=[
            pltpu.VMEM((CHUNK,), jnp.int32),
            pltpu.VMEM((CHUNK,), jnp.float32),
            pltpu.VMEM((BPW,), jnp.float32),
            pltpu.SemaphoreType.DMA,
        ],
    )
    def sc_kernel(w_hbm, out_hbm, idx_v, rows_v, acc_v, sem):
        wid = lax.axis_index("s") * NC + lax.axis_index("c")

        def mk(i, c):
            idx_v[pl.ds(i * L, L)] = lax.iota(jnp.int32, L) + (wid * CHUNK + i * L)
            return c

        lax.fori_loop(0, CHUNK // L, mk, 0)
        pltpu.async_copy(w_hbm.at[idx_v], rows_v, sem).wait()

        def accum(j, c):
            a = jnp.zeros((L,), jnp.float32)
            for f in range(F):
                a = a + rows_v[pl.ds(f * BPW + j * L, L)]
            acc_v[pl.ds(j * L, L)] = a
            return c

        lax.fori_loop(0, NJ, accum, 0)
        pltpu.sync_copy(acc_v, out_hbm.at[pl.ds(wid * BPW, BPW)])

    return sc_kernel(wf)


# speedup vs baseline: 2.1667x; 2.1667x over previous
import functools

import jax
import jax.numpy as jnp
from jax import lax
from jax.experimental import pallas as pl
from jax.experimental.pallas import tpu as pltpu
from jax.experimental.pallas import tpu_sc as plsc

F = 26
B = 16384
FD = 100000
NC, NS, L = 2, 16, 16
NW = NC * NS
BPW = B // NW
CHUNK = F * BPW
NJ = BPW // L


def kernel(x, W, bias):
    xf = x.reshape(-1)
    wf = jnp.pad(W, ((0, 960), (0, 0))).reshape(-1)
    b16 = jnp.broadcast_to(bias.astype(jnp.float32), (L,))

    mesh = plsc.VectorSubcoreMesh(core_axis_name="c", subcore_axis_name="s")

    @functools.partial(
        pl.kernel,
        mesh=mesh,
        out_type=jax.ShapeDtypeStruct((B,), jnp.float32),
        compiler_params=pltpu.CompilerParams(needs_layout_passes=False),
        scratch_types=[
            pltpu.VMEM((CHUNK,), jnp.int32),
            pltpu.VMEM((CHUNK,), jnp.int32),
            pltpu.VMEM((CHUNK,), jnp.float32),
            pltpu.VMEM((L,), jnp.float32),
            pltpu.VMEM((BPW,), jnp.float32),
            pltpu.SemaphoreType.DMA,
        ],
    )
    def sc_kernel(x_hbm, w_hbm, b_hbm, out_hbm, xv, idx_v, rows_v, bias_v, acc_v, sem):
        wid = lax.axis_index("s") * NC + lax.axis_index("c")
        pltpu.sync_copy(x_hbm.at[pl.ds(wid * CHUNK, CHUNK)], xv)
        pltpu.sync_copy(b_hbm, bias_v)

        lane26 = lax.iota(jnp.int32, L) * F

        def mk_idx(j, carry):
            jbase = j * L * F
            for f in range(F):
                g = plsc.load_gather(xv, [lane26 + (jbase + f)])
                idx_v[pl.ds(f * BPW + j * L, L)] = g + f * FD
            return carry

        lax.fori_loop(0, NJ, mk_idx, 0)

        pltpu.async_copy(w_hbm.at[idx_v], rows_v, sem).wait()

        def accum(j, carry):
            a = bias_v[...]
            for f in range(F):
                a = a + rows_v[pl.ds(f * BPW + j * L, L)]
            acc_v[pl.ds(j * L, L)] = 1.0 / (1.0 + jnp.exp(-a))
            return carry

        lax.fori_loop(0, NJ, accum, 0)

        pltpu.sync_copy(acc_v, out_hbm.at[pl.ds(wid * BPW, BPW)])

    return sc_kernel(xf, wf, b16)


# speedup vs baseline: 2.7836x; 1.2847x over previous
import functools

import jax
import jax.numpy as jnp
from jax import lax
from jax.experimental import pallas as pl
from jax.experimental.pallas import tpu as pltpu
from jax.experimental.pallas import tpu_sc as plsc

F = 26
B = 16384
FD = 100000
NC, NS, L = 2, 16, 16
NW = NC * NS
BPW = B // NW
CHUNK = F * BPW
NJ = BPW // L


def kernel(x, W, bias):
    xt = jnp.swapaxes(x, 0, 1)
    wf = jnp.pad(W, ((0, 960), (0, 0))).reshape(-1)
    b16 = jnp.broadcast_to(bias.astype(jnp.float32), (L,))

    mesh = plsc.VectorSubcoreMesh(core_axis_name="c", subcore_axis_name="s")

    @functools.partial(
        pl.kernel,
        mesh=mesh,
        out_type=jax.ShapeDtypeStruct((B,), jnp.float32),
        compiler_params=pltpu.CompilerParams(needs_layout_passes=False),
        scratch_types=[
            pltpu.VMEM((F, BPW), jnp.int32),
            pltpu.VMEM((CHUNK,), jnp.int32),
            pltpu.VMEM((CHUNK,), jnp.float32),
            pltpu.VMEM((L,), jnp.float32),
            pltpu.VMEM((BPW,), jnp.float32),
            pltpu.SemaphoreType.DMA,
        ],
    )
    def sc_kernel(x_hbm, w_hbm, b_hbm, out_hbm, xv, idx_v, rows_v, bias_v, acc_v, sem):
        wid = lax.axis_index("s") * NC + lax.axis_index("c")
        pltpu.sync_copy(x_hbm.at[:, pl.ds(wid * BPW, BPW)], xv)
        pltpu.sync_copy(b_hbm, bias_v)

        def mk_idx(j, carry):
            for f in range(F):
                idx_v[pl.ds(f * BPW + j * L, L)] = xv[f, pl.ds(j * L, L)] + f * FD
            return carry

        lax.fori_loop(0, NJ, mk_idx, 0)

        pltpu.async_copy(w_hbm.at[idx_v], rows_v, sem).wait()

        def accum(j, carry):
            a = bias_v[...]
            for f in range(F):
                a = a + rows_v[pl.ds(f * BPW + j * L, L)]
            acc_v[pl.ds(j * L, L)] = 1.0 / (1.0 + jnp.exp(-a))
            return carry

        lax.fori_loop(0, NJ, accum, 0)

        pltpu.sync_copy(acc_v, out_hbm.at[pl.ds(wid * BPW, BPW)])

    return sc_kernel(xt, wf, b16)


# speedup vs baseline: 2.8343x; 1.0182x over previous
import functools

import jax
import jax.numpy as jnp
from jax import lax
import numpy as np
from jax.experimental import pallas as pl
from jax.experimental.pallas import tpu as pltpu
from jax.experimental.pallas import tpu_sc as plsc

F = 26
B = 16384
FD = 100000
NC, NS, L = 2, 16, 16
NW = NC * NS
BPW = B // NW
CHUNK = F * BPW
NJ = BPW // L
SPLITS = ((0, 13), (13, 13))


def kernel(x, W, bias):
    offsets = jnp.asarray(np.arange(F, dtype=np.int32) * FD)
    xt = jnp.swapaxes(x, 0, 1) + offsets[:, None]
    wf = jnp.pad(W, ((0, 960), (0, 0))).reshape(-1)
    b16 = jnp.broadcast_to(bias.astype(jnp.float32), (L,))

    mesh = plsc.VectorSubcoreMesh(core_axis_name="c", subcore_axis_name="s")

    @functools.partial(
        pl.kernel,
        mesh=mesh,
        out_type=jax.ShapeDtypeStruct((B,), jnp.float32),
        compiler_params=pltpu.CompilerParams(needs_layout_passes=False),
        scratch_types=[
            pltpu.VMEM((CHUNK,), jnp.int32),
            pltpu.VMEM((CHUNK,), jnp.float32),
            pltpu.VMEM((L,), jnp.float32),
            pltpu.VMEM((BPW,), jnp.float32),
            pltpu.SemaphoreType.DMA,
            pltpu.SemaphoreType.DMA,
        ],
    )
    def sc_kernel(x_hbm, w_hbm, b_hbm, out_hbm, idx_v, rows_v, bias_v, acc_v, sem, sem2):
        wid = lax.axis_index("s") * NC + lax.axis_index("c")
        b0 = wid * BPW
        idx_copies = [
            pltpu.async_copy(
                x_hbm.at[f, pl.ds(b0, BPW)], idx_v.at[pl.ds(f * BPW, BPW)], sem2
            )
            for f in range(F)
        ]
        pltpu.sync_copy(b_hbm, bias_v)
        for c in idx_copies:
            c.wait()

        copies = [
            pltpu.async_copy(
                w_hbm.at[idx_v.at[pl.ds(s * (CHUNK // 2), CHUNK // 2)]],
                rows_v.at[pl.ds(s * (CHUNK // 2), CHUNK // 2)],
                sem,
            )
            for s in range(2)
        ]
        for c in copies:
            c.wait()

        def accum(j, carry):
            a = bias_v[...]
            for f in range(F):
                a = a + rows_v[pl.ds(f * BPW + j * L, L)]
            acc_v[pl.ds(j * L, L)] = 1.0 / (1.0 + jnp.exp(-a))
            return carry

        lax.fori_loop(0, NJ, accum, 0)

        pltpu.sync_copy(acc_v, out_hbm.at[pl.ds(wid * BPW, BPW)])

    return sc_kernel(xt, wf, b16)


# speedup vs baseline: 2.8355x; 1.0004x over previous
import functools

import jax
import jax.numpy as jnp
from jax import lax
import numpy as np
from jax.experimental import pallas as pl
from jax.experimental.pallas import tpu as pltpu
from jax.experimental.pallas import tpu_sc as plsc

F = 26
B = 16384
FD = 100000
NC, NS, L = 2, 16, 16
NW = NC * NS
BPW = B // NW
CHUNK = F * BPW
NJ = BPW // L
SPLITS = ((0, 13), (13, 13))


def kernel(x, W, bias):
    offsets = jnp.asarray(np.arange(F, dtype=np.int32) * FD)
    xt = jnp.swapaxes(x, 0, 1) + offsets[:, None]
    wf = jnp.pad(jnp.swapaxes(W, 0, 1), ((0, 0), (0, 960))).reshape(-1)
    b16 = jnp.broadcast_to(bias.astype(jnp.float32), (L,))

    mesh = plsc.VectorSubcoreMesh(core_axis_name="c", subcore_axis_name="s")

    @functools.partial(
        pl.kernel,
        mesh=mesh,
        out_type=jax.ShapeDtypeStruct((B,), jnp.float32),
        compiler_params=pltpu.CompilerParams(needs_layout_passes=False),
        scratch_types=[
            pltpu.VMEM((CHUNK,), jnp.int32),
            pltpu.VMEM((CHUNK,), jnp.float32),
            pltpu.VMEM((L,), jnp.float32),
            pltpu.VMEM((BPW,), jnp.float32),
            pltpu.SemaphoreType.DMA,
            pltpu.SemaphoreType.DMA,
        ],
    )
    def sc_kernel(x_hbm, w_hbm, b_hbm, out_hbm, idx_v, rows_v, bias_v, acc_v, sem, sem2):
        wid = lax.axis_index("s") * NC + lax.axis_index("c")
        b0 = wid * BPW
        idx_copies = [
            pltpu.async_copy(
                x_hbm.at[f, pl.ds(b0, BPW)], idx_v.at[pl.ds(f * BPW, BPW)], sem2
            )
            for f in range(F)
        ]
        pltpu.sync_copy(b_hbm, bias_v)
        for c in idx_copies:
            c.wait()

        copies = [
            pltpu.async_copy(
                w_hbm.at[idx_v.at[pl.ds(s * (CHUNK // 2), CHUNK // 2)]],
                rows_v.at[pl.ds(s * (CHUNK // 2), CHUNK // 2)],
                sem,
            )
            for s in range(2)
        ]
        for c in copies:
            c.wait()

        def accum(j, carry):
            a = bias_v[...]
            for f in range(F):
                a = a + rows_v[pl.ds(f * BPW + j * L, L)]
            acc_v[pl.ds(j * L, L)] = 1.0 / (1.0 + jnp.exp(-a))
            return carry

        lax.fori_loop(0, NJ, accum, 0)

        pltpu.sync_copy(acc_v, out_hbm.at[pl.ds(wid * BPW, BPW)])

    return sc_kernel(xt, wf, b16)


# speedup vs baseline: 2.8357x; 1.0000x over previous
import functools

import jax
import jax.numpy as jnp
from jax import lax
import numpy as np
from jax.experimental import pallas as pl
from jax.experimental.pallas import tpu as pltpu
from jax.experimental.pallas import tpu_sc as plsc

F = 26
B = 16384
FD = 100000
NC, NS, L = 2, 16, 16
NW = NC * NS
BPW = B // NW
CHUNK = F * BPW
NJ = BPW // L
SPLITS = ((0, 13), (13, 13))


def kernel(x, W, bias):
    offsets = jnp.asarray(np.arange(F, dtype=np.int32) * FD)
    xt = jnp.swapaxes(x, 0, 1) + offsets[:, None]
    wf = jnp.concatenate([W, jnp.zeros((960, 1), jnp.float32)], axis=0).reshape(-1)
    b16 = jnp.broadcast_to(bias.astype(jnp.float32), (L,))

    mesh = plsc.VectorSubcoreMesh(core_axis_name="c", subcore_axis_name="s")

    @functools.partial(
        pl.kernel,
        mesh=mesh,
        out_type=jax.ShapeDtypeStruct((B,), jnp.float32),
        compiler_params=pltpu.CompilerParams(needs_layout_passes=False),
        scratch_types=[
            pltpu.VMEM((CHUNK,), jnp.int32),
            pltpu.VMEM((CHUNK,), jnp.float32),
            pltpu.VMEM((L,), jnp.float32),
            pltpu.VMEM((BPW,), jnp.float32),
            pltpu.SemaphoreType.DMA,
            pltpu.SemaphoreType.DMA,
        ],
    )
    def sc_kernel(x_hbm, w_hbm, b_hbm, out_hbm, idx_v, rows_v, bias_v, acc_v, sem, sem2):
        wid = lax.axis_index("s") * NC + lax.axis_index("c")
        b0 = wid * BPW
        idx_copies = [
            pltpu.async_copy(
                x_hbm.at[f, pl.ds(b0, BPW)], idx_v.at[pl.ds(f * BPW, BPW)], sem2
            )
            for f in range(F)
        ]
        pltpu.sync_copy(b_hbm, bias_v)
        for c in idx_copies:
            c.wait()

        copies = [
            pltpu.async_copy(
                w_hbm.at[idx_v.at[pl.ds(s * (CHUNK // 2), CHUNK // 2)]],
                rows_v.at[pl.ds(s * (CHUNK // 2), CHUNK // 2)],
                sem,
            )
            for s in range(2)
        ]
        for c in copies:
            c.wait()

        def accum(j, carry):
            a = bias_v[...]
            for f in range(F):
                a = a + rows_v[pl.ds(f * BPW + j * L, L)]
            acc_v[pl.ds(j * L, L)] = 1.0 / (1.0 + jnp.exp(-a))
            return carry

        lax.fori_loop(0, NJ, accum, 0)

        pltpu.sync_copy(acc_v, out_hbm.at[pl.ds(wid * BPW, BPW)])

    return sc_kernel(xt, wf, b16)
